# baseline (speedup 1.0000x reference)
import functools

import jax
import jax.numpy as jnp
from jax import lax
from jax.experimental import pallas as pl
from jax.experimental.pallas import tpu as pltpu
from jax.experimental.pallas import tpu_sc as plsc

D = 64
K = 20
CH = 16
LANES = 16
NSPLIT = 4


@functools.lru_cache(maxsize=None)
def _make_sc_kernel(B, V):
    info = plsc.get_sparse_core_info()
    NC, NS = info.num_cores, info.num_subcores
    NW = NC * NS
    bpw = B // NW
    nch = bpw // CH
    npg = CH * K // NSPLIT
    orow = bpw // 8
    f32 = jnp.float32
    i32 = jnp.int32
    mesh = plsc.VectorSubcoreMesh(core_axis_name="c", subcore_axis_name="s")

    @functools.partial(
        pl.kernel,
        out_type=(
            jax.ShapeDtypeStruct((B // 8, 128), f32),
            jax.ShapeDtypeStruct((B // 8, 128), f32),
        ),
        mesh=mesh,
        scratch_types=[
            pltpu.VMEM((bpw,), i32),
            pltpu.VMEM((bpw,), i32),
            pltpu.VMEM((bpw * K,), i32),
            pltpu.VMEM((2, CH, 128), f32),
            pltpu.VMEM((2, CH, 128), f32),
            pltpu.VMEM((2, CH * K, 128), f32),
            pltpu.VMEM((orow, 128), f32),
            pltpu.VMEM((orow, 128), f32),
            pltpu.SemaphoreType.DMA,
            pltpu.SemaphoreType.DMA,
        ],
    )
    def sc_kernel(tgt_hbm, ctx_hbm, negf_hbm, up_hbm, vp_hbm,
                  pos_hbm, negp_hbm,
                  tgt_i, ctx_i, neg_i,
                  urows, vrows, nrows,
                  pos_pv, neg_pv,
                  rsem0, rsem1):
        wid = lax.axis_index("s") * NC + lax.axis_index("c")
        base = wid * bpw
        rsems = (rsem0, rsem1)

        pltpu.sync_copy(tgt_hbm.at[pl.ds(base, bpw)], tgt_i)
        pltpu.sync_copy(ctx_hbm.at[pl.ds(base, bpw)], ctx_i)
        pltpu.sync_copy(negf_hbm.at[pl.ds(base * K, bpw * K)], neg_i)

        def fire_rows(c, s):
            sem = rsems[s]
            cb = c * CH
            pltpu.async_copy(up_hbm.at[tgt_i.at[pl.ds(cb, CH)]],
                             urows.at[s], sem)
            pltpu.async_copy(vp_hbm.at[ctx_i.at[pl.ds(cb, CH)]],
                             vrows.at[s], sem)
            for j in range(NSPLIT):
                pltpu.async_copy(
                    vp_hbm.at[neg_i.at[pl.ds(cb * K + j * npg, npg)]],
                    nrows.at[s, pl.ds(j * npg, npg)], sem)

        def wait_rows(s):
            sem = rsems[s]
            pltpu.make_async_copy(up_hbm.at[pl.ds(0, CH)], urows.at[s],
                                  sem).wait()
            pltpu.make_async_copy(vp_hbm.at[pl.ds(0, CH)], vrows.at[s],
                                  sem).wait()
            for j in range(NSPLIT):
                pltpu.make_async_copy(
                    vp_hbm.at[pl.ds(0, npg)],
                    nrows.at[s, pl.ds(j * npg, npg)], sem).wait()

        def compute(c, s):
            def body(i, carry):
                row = c * CH + i
                rb = i * K
                acc = [nrows[s, rb, pl.ds(16 * d, 16)]
                       for d in range(4)]
                for k in range(1, K):
                    for d in range(4):
                        acc[d] = acc[d] + nrows[s, rb + k,
                                                pl.ds(16 * d, 16)]
                pv = None
                nv = None
                for d in range(4):
                    u = urows[s, i, pl.ds(16 * d, 16)]
                    v = vrows[s, i, pl.ds(16 * d, 16)]
                    pv = u * v if pv is None else pv + u * v
                    nv = u * acc[d] if nv is None else nv + u * acc[d]
                r8 = row // 8
                lo = (row % 8) * LANES
                pos_pv[r8, pl.ds(lo, LANES)] = pv
                neg_pv[r8, pl.ds(lo, LANES)] = nv
                return carry

            lax.fori_loop(0, CH, body, 0)

        fire_rows(0, 0)

        def chunk_pair(cc, carry):
            c0 = cc * 2
            for t in range(2):
                c = c0 + t

                @pl.when(c + 1 < nch)
                def _():
                    fire_rows(c + 1, 1 - t)

                wait_rows(t)
                compute(c, t)
            return carry

        lax.fori_loop(0, nch // 2, chunk_pair, 0)

        pltpu.sync_copy(pos_pv, pos_hbm.at[pl.ds(wid * orow, orow)])
        pltpu.sync_copy(neg_pv, negp_hbm.at[pl.ds(wid * orow, orow)])

    return sc_kernel


def _finish_body(pos_ref, neg_ref, out_ref):
    lanes = lax.broadcasted_iota(jnp.int32, (128, 8), 0)
    segs = lax.broadcasted_iota(jnp.int32, (128, 8), 1)
    mseg = jnp.where(lanes // LANES == segs, 1.0, 0.0).astype(jnp.float32)
    p = jnp.dot(pos_ref[...], mseg)
    n = jnp.dot(neg_ref[...], mseg)
    loss = jax.nn.log_sigmoid(p) + jax.nn.log_sigmoid(-n)
    nb = pos_ref.shape[0] * 8
    total = -jnp.sum(loss) * (1.0 / nb)
    out_ref[...] = total * jnp.ones((1, 1), jnp.float32)


@jax.jit
def kernel(target, context, neg, u_emb, v_emb):
    B, Kn = neg.shape
    V, Dn = u_emb.shape
    assert Kn == K and Dn == D
    tgt = target.astype(jnp.int32)
    ctx = context.astype(jnp.int32)
    negf = neg.astype(jnp.int32).reshape(B * K)
    up = jnp.pad(u_emb, ((0, 0), (0, D)))
    vp = jnp.pad(v_emb, ((0, 0), (0, D)))
    pos_pv, neg_pv = _make_sc_kernel(B, V)(tgt, ctx, negf, up, vp)
    out = pl.pallas_call(
        _finish_body,
        out_shape=jax.ShapeDtypeStruct((1, 1), jnp.float32),
    )(pos_pv, neg_pv)
    return out[0, 0]


# speedup vs baseline: 1.1289x; 1.1289x over previous
import functools

import jax
import jax.numpy as jnp
from jax import lax
from jax.experimental import pallas as pl
from jax.experimental.pallas import tpu as pltpu
from jax.experimental.pallas import tpu_sc as plsc

D = 64
K = 20
CH = 16
LANES = 16
NSPLIT = 4


@functools.lru_cache(maxsize=None)
def _make_sc_kernel(B, V):
    info = plsc.get_sparse_core_info()
    NC, NS = info.num_cores, info.num_subcores
    NW = NC * NS
    bpw = B // NW
    nch = bpw // CH
    npg = CH * K // NSPLIT
    orow = bpw // 8
    f32 = jnp.float32
    i32 = jnp.int32
    mesh = plsc.VectorSubcoreMesh(core_axis_name="c", subcore_axis_name="s")

    @functools.partial(
        pl.kernel,
        out_type=(
            jax.ShapeDtypeStruct((B // 8, 128), f32),
            jax.ShapeDtypeStruct((B // 8, 128), f32),
        ),
        mesh=mesh,
        scratch_types=[
            pltpu.VMEM((bpw,), i32),
            pltpu.VMEM((bpw,), i32),
            pltpu.VMEM((bpw * K,), i32),
            pltpu.VMEM((2, CH, 128), f32),
            pltpu.VMEM((2, CH, 128), f32),
            pltpu.VMEM((2, CH * K, 128), f32),
            pltpu.VMEM((orow, 128), f32),
            pltpu.VMEM((orow, 128), f32),
            pltpu.SemaphoreType.DMA,
            pltpu.SemaphoreType.DMA,
        ],
    )
    def sc_kernel(tgt_hbm, ctx_hbm, negf_hbm, tab_hbm,
                  pos_hbm, negp_hbm,
                  tgt_i, ctx_i, neg_i,
                  urows, vrows, nrows,
                  pos_pv, neg_pv,
                  rsem0, rsem1):
        wid = lax.axis_index("s") * NC + lax.axis_index("c")
        base = wid * bpw
        rsems = (rsem0, rsem1)

        pltpu.sync_copy(tgt_hbm.at[pl.ds(base, bpw)], tgt_i)
        pltpu.sync_copy(ctx_hbm.at[pl.ds(base, bpw)], ctx_i)
        pltpu.sync_copy(negf_hbm.at[pl.ds(base * K, bpw * K)], neg_i)

        def fire_rows(c, s):
            sem = rsems[s]
            cb = c * CH
            pltpu.async_copy(tab_hbm.at[tgt_i.at[pl.ds(cb, CH)]],
                             urows.at[s], sem)
            pltpu.async_copy(tab_hbm.at[ctx_i.at[pl.ds(cb, CH)]],
                             vrows.at[s], sem)
            for j in range(NSPLIT):
                pltpu.async_copy(
                    tab_hbm.at[neg_i.at[pl.ds(cb * K + j * npg, npg)]],
                    nrows.at[s, pl.ds(j * npg, npg)], sem)

        def wait_rows(s):
            sem = rsems[s]
            pltpu.make_async_copy(tab_hbm.at[pl.ds(0, CH)], urows.at[s],
                                  sem).wait()
            pltpu.make_async_copy(tab_hbm.at[pl.ds(0, CH)], vrows.at[s],
                                  sem).wait()
            for j in range(NSPLIT):
                pltpu.make_async_copy(
                    tab_hbm.at[pl.ds(0, npg)],
                    nrows.at[s, pl.ds(j * npg, npg)], sem).wait()

        def compute(c, s):
            def body(i, carry):
                row = c * CH + i
                rb = i * K
                acc = [nrows[s, rb, pl.ds(D + 16 * d, 16)]
                       for d in range(4)]
                for k in range(1, K):
                    for d in range(4):
                        acc[d] = acc[d] + nrows[s, rb + k,
                                                pl.ds(D + 16 * d, 16)]
                pv = None
                nv = None
                for d in range(4):
                    u = urows[s, i, pl.ds(16 * d, 16)]
                    v = vrows[s, i, pl.ds(D + 16 * d, 16)]
                    pv = u * v if pv is None else pv + u * v
                    nv = u * acc[d] if nv is None else nv + u * acc[d]
                r8 = row // 8
                lo = (row % 8) * LANES
                pos_pv[r8, pl.ds(lo, LANES)] = pv
                neg_pv[r8, pl.ds(lo, LANES)] = nv
                return carry

            lax.fori_loop(0, CH, body, 0)

        fire_rows(0, 0)

        def chunk_pair(cc, carry):
            c0 = cc * 2
            for t in range(2):
                c = c0 + t

                @pl.when(c + 1 < nch)
                def _():
                    fire_rows(c + 1, 1 - t)

                wait_rows(t)
                compute(c, t)
            return carry

        lax.fori_loop(0, nch // 2, chunk_pair, 0)

        pltpu.sync_copy(pos_pv, pos_hbm.at[pl.ds(wid * orow, orow)])
        pltpu.sync_copy(neg_pv, negp_hbm.at[pl.ds(wid * orow, orow)])

    return sc_kernel


def _finish_body(pos_ref, neg_ref, out_ref):
    lanes = lax.broadcasted_iota(jnp.int32, (128, 8), 0)
    segs = lax.broadcasted_iota(jnp.int32, (128, 8), 1)
    mseg = jnp.where(lanes // LANES == segs, 1.0, 0.0).astype(jnp.float32)
    p = jnp.dot(pos_ref[...], mseg)
    n = jnp.dot(neg_ref[...], mseg)
    loss = jax.nn.log_sigmoid(p) + jax.nn.log_sigmoid(-n)
    nb = pos_ref.shape[0] * 8
    total = -jnp.sum(loss) * (1.0 / nb)
    out_ref[...] = total * jnp.ones((1, 1), jnp.float32)


@jax.jit
def kernel(target, context, neg, u_emb, v_emb):
    B, Kn = neg.shape
    V, Dn = u_emb.shape
    assert Kn == K and Dn == D
    tgt = target.astype(jnp.int32)
    ctx = context.astype(jnp.int32)
    negf = neg.astype(jnp.int32).reshape(B * K)
    tab = jnp.concatenate([u_emb, v_emb], axis=1)
    pos_pv, neg_pv = _make_sc_kernel(B, V)(tgt, ctx, negf, tab)
    out = pl.pallas_call(
        _finish_body,
        out_shape=jax.ShapeDtypeStruct((1, 1), jnp.float32),
    )(pos_pv, neg_pv)
    return out[0, 0]
